# baseline (speedup 1.0000x reference)
import jax
import jax.numpy as jnp
from jax import lax
from jax.experimental import pallas as pl
from jax.experimental.pallas import tpu as pltpu
from jax.experimental.pallas import tpu_sc as plsc

N_NODES = 10000
D_FEAT = 128
N_EDGES = 320000

NC = 2
NS = 16
HALF = D_FEAT // NC
CHUNK = 128
GCHUNK = 256
NBUF = 2
PHASES = 4
PEDGES = 5120
NG = PEDGES // GCHUNK
NSCC = PEDGES // CHUNK
EPT = PHASES * PEDGES
BP = NS * EPT
NSCHUNKS = BP // CHUNK
XS_ROWS = 10016
ACC_ROWS = 10240
ZROWS = ACC_ROWS // NS
OROWS = 624
OROWS_LAST = N_NODES - 15 * OROWS


def _sc_body(x0h, x1h, idxjh, idxih, outh, idxj_v, idxi_v, rows_v, zbuf,
             xs, acc, gs0, gs1, ss0, ss1):
    c = lax.axis_index("c")
    s = lax.axis_index("s")
    cbase = c * HALF
    gsem = (gs0, gs1)
    ssem = (ss0, ss1)

    def slot(b):
        return rows_v.at[pl.ds(b * GCHUNK, GCHUNK)]

    xr0 = s * 640

    def stage_block(r0, n):
        @pl.when(c == 0)
        def _():
            pltpu.sync_copy(x0h.at[pl.ds(r0, n)], rows_v.at[pl.ds(0, n)])

        @pl.when(c != 0)
        def _():
            pltpu.sync_copy(x1h.at[pl.ds(r0, n)], rows_v.at[pl.ds(0, n)])

        pltpu.sync_copy(rows_v.at[pl.ds(0, n)], xs.at[pl.ds(r0, n)])

    @pl.when(s != NS - 1)
    def _():
        for blk in range(2):
            stage_block(xr0 + blk * 320, 320)

    @pl.when(s == NS - 1)
    def _():
        stage_block(xr0, 320)
        stage_block(xr0 + 320, 80)

    zv = jnp.zeros((16,), jnp.float32)
    for r in range(16):
        for q in range(HALF // 16):
            zbuf[r, pl.ds(q * 16, 16)] = zv

    def zbody(i, carry):
        pltpu.sync_copy(zbuf, acc.at[pl.ds(s * ZROWS + i * 16, 16)])
        return carry

    lax.fori_loop(0, ZROWS // 16, zbody, 0)
    plsc.subcore_barrier()

    def issue_gather(ch, b):
        pltpu.async_copy(
            xs.at[idxj_v.at[pl.ds(ch * GCHUNK, GCHUNK)]], slot(b), gsem[b])

    def drain(sem, b):
        pltpu.make_async_copy(x0h.at[pl.ds(0, GCHUNK)], slot(b), sem).wait()

    def phase(p, carry):
        pltpu.sync_copy(
            idxjh.at[pl.ds(s * EPT + p * PEDGES, PEDGES)], idxj_v)
        pltpu.sync_copy(
            idxih.at[pl.ds(s * (EPT // CHUNK) + p * NSCC, NSCC)], idxi_v)

        for b in range(NBUF):
            issue_gather(b, b)

        def mbody(i, carry2):
            g = i * NBUF
            for b in range(NBUF):
                ch = g + b
                drain(gsem[b], b)
                for q in range(GCHUNK // CHUNK):
                    pltpu.async_copy(
                        rows_v.at[pl.ds(b * GCHUNK + q * CHUNK, CHUNK)],
                        acc.at[idxi_v.at[ch * (GCHUNK // CHUNK) + q]],
                        ssem[b], add=True)
            for b in range(NBUF):
                ch = g + b
                drain(ssem[b], b)

                @pl.when(ch + NBUF < NG)
                def _():
                    issue_gather(ch + NBUF, b)
            return carry2

        lax.fori_loop(0, NG // NBUF, mbody, 0)
        return carry

    lax.fori_loop(0, PHASES, phase, 0)
    plsc.subcore_barrier()

    r0 = s * OROWS

    @pl.when(s != NS - 1)
    def _():
        pltpu.sync_copy(acc.at[pl.ds(r0, OROWS)],
                        outh.at[pl.ds(r0, OROWS), pl.ds(cbase, HALF)])

    @pl.when(s == NS - 1)
    def _():
        pltpu.sync_copy(acc.at[pl.ds(r0, OROWS_LAST)],
                        outh.at[pl.ds(r0, OROWS_LAST), pl.ds(cbase, HALF)])


@jax.jit
def _sc_call(x0, x1, idxj, idxi):
    mesh = plsc.VectorSubcoreMesh(core_axis_name="c", subcore_axis_name="s")
    return pl.kernel(
        _sc_body,
        out_type=jax.ShapeDtypeStruct((N_NODES, D_FEAT), jnp.float32),
        mesh=mesh,
        compiler_params=pltpu.CompilerParams(use_tc_tiling_on_sc=False),
        scratch_types=[
            pltpu.VMEM((PEDGES,), jnp.int32),
            pltpu.VMEM((NSCC, CHUNK), jnp.int32),
            pltpu.VMEM((NBUF * GCHUNK, HALF), jnp.float32),
            pltpu.VMEM((16, HALF), jnp.float32),
            pltpu.VMEM_SHARED((XS_ROWS, HALF), jnp.float32),
            pltpu.VMEM_SHARED((ACC_ROWS, HALF), jnp.float32),
        ] + [pltpu.SemaphoreType.DMA] * 4,
    )(x0, x1, idxj, idxi)


def kernel(x, a3_indices, e3):
    del e3
    idx_j = a3_indices[:, 1]
    idx_i = a3_indices[:, 2]
    pad = BP - N_EDGES
    idx_j = jnp.concatenate([idx_j, jnp.zeros((pad,), jnp.int32)])
    idx_i = jnp.concatenate(
        [idx_i, jnp.full((pad,), N_NODES, jnp.int32)]).reshape(NSCHUNKS, CHUNK)
    return _sc_call(x[:, :HALF], x[:, HALF:], idx_j, idx_i)


# speedup vs baseline: 1.0769x; 1.0769x over previous
import jax
import jax.numpy as jnp
from jax import lax
from jax.experimental import pallas as pl
from jax.experimental.pallas import tpu as pltpu
from jax.experimental.pallas import tpu_sc as plsc

N_NODES = 10000
D_FEAT = 128
N_EDGES = 320000

NC = 2
NS = 16
HALF = D_FEAT // NC
CHUNK = 128
NBUF = 5
PHASES = 8
PEDGES = 2560
NG = PEDGES // CHUNK
EPT = PHASES * PEDGES
BP = NS * EPT
NSCHUNKS = BP // CHUNK
XS_ROWS = 10016
ACC_ROWS = 10240
ZROWS = ACC_ROWS // NS
OROWS = 624
OROWS_LAST = N_NODES - 15 * OROWS


def _sc_body(x0h, x1h, idxjh, idxih, outh, idxj_v, idxi_v, rows_v, zbuf,
             xs, acc, gs0, gs1, gs2, gs3, gs4, ss0, ss1, ss2, ss3, ss4):
    c = lax.axis_index("c")
    s = lax.axis_index("s")
    cbase = c * HALF
    gsem = (gs0, gs1, gs2, gs3, gs4)
    ssem = (ss0, ss1, ss2, ss3, ss4)

    def slot(b, n=CHUNK):
        return rows_v.at[pl.ds(b * CHUNK, n)]

    xr0 = s * 640
    nlast = 400 - 3 * CHUNK

    def stage_issue(blk, n):
        r0 = xr0 + blk * CHUNK

        @pl.when(c == 0)
        def _():
            pltpu.async_copy(x0h.at[pl.ds(r0, n)], slot(blk, n), gsem[blk])

        @pl.when(c != 0)
        def _():
            pltpu.async_copy(x1h.at[pl.ds(r0, n)], slot(blk, n), gsem[blk])

    def stage_push(blk, n):
        r0 = xr0 + blk * CHUNK
        pltpu.make_async_copy(x0h.at[pl.ds(0, n)], slot(blk, n),
                              gsem[blk]).wait()
        pltpu.async_copy(slot(blk, n), xs.at[pl.ds(r0, n)], ssem[blk])

    def stage_drain(blk, n):
        pltpu.make_async_copy(x0h.at[pl.ds(0, n)], slot(blk, n),
                              ssem[blk]).wait()

    @pl.when(s != NS - 1)
    def _():
        for blk in range(5):
            stage_issue(blk, CHUNK)
        for blk in range(5):
            stage_push(blk, CHUNK)
        for blk in range(5):
            stage_drain(blk, CHUNK)

    @pl.when(s == NS - 1)
    def _():
        for blk in range(3):
            stage_issue(blk, CHUNK)
        stage_issue(3, nlast)
        for blk in range(3):
            stage_push(blk, CHUNK)
        stage_push(3, nlast)
        for blk in range(3):
            stage_drain(blk, CHUNK)
        stage_drain(3, nlast)

    zv = jnp.zeros((16,), jnp.float32)
    for r in range(16):
        for q in range(HALF // 16):
            zbuf[r, pl.ds(q * 16, 16)] = zv

    def zbody(i, carry):
        for b in range(NBUF):
            pltpu.async_copy(
                zbuf, acc.at[pl.ds(s * ZROWS + (i * NBUF + b) * 16, 16)],
                ssem[b])
        return carry

    lax.fori_loop(0, ZROWS // (16 * NBUF), zbody, 0)
    for b in range(NBUF):
        pltpu.make_async_copy(x0h.at[pl.ds(0, CHUNK)], slot(b),
                              ssem[b]).wait()
    plsc.subcore_barrier()

    def issue_gather(ch, b):
        pltpu.async_copy(
            xs.at[idxj_v.at[pl.ds(ch * CHUNK, CHUNK)]], slot(b), gsem[b])

    def drain(sem, b):
        pltpu.make_async_copy(x0h.at[pl.ds(0, CHUNK)], slot(b), sem).wait()

    def phase(p, carry):
        pltpu.async_copy(
            idxjh.at[pl.ds(s * EPT + p * PEDGES, PEDGES)], idxj_v, gsem[0])
        pltpu.async_copy(
            idxih.at[pl.ds(s * (EPT // CHUNK) + p * NG, NG)], idxi_v, gsem[1])
        pltpu.make_async_copy(
            idxjh.at[pl.ds(0, PEDGES)], idxj_v, gsem[0]).wait()
        pltpu.make_async_copy(
            idxih.at[pl.ds(0, NG)], idxi_v, gsem[1]).wait()

        for b in range(NBUF):
            issue_gather(b, b)

        def mbody(i, carry2):
            g = i * NBUF
            for b in range(NBUF):
                drain(gsem[b], b)
                pltpu.async_copy(
                    slot(b), acc.at[idxi_v.at[g + b]], ssem[b], add=True)
            for b in range(NBUF):
                drain(ssem[b], b)

                @pl.when(g + b + NBUF < NG)
                def _():
                    issue_gather(g + b + NBUF, b)
            return carry2

        lax.fori_loop(0, NG // NBUF, mbody, 0)
        return carry

    lax.fori_loop(0, PHASES, phase, 0)
    plsc.subcore_barrier()

    r0 = s * OROWS

    @pl.when(s != NS - 1)
    def _():
        pltpu.sync_copy(acc.at[pl.ds(r0, OROWS)],
                        outh.at[pl.ds(r0, OROWS), pl.ds(cbase, HALF)])

    @pl.when(s == NS - 1)
    def _():
        pltpu.sync_copy(acc.at[pl.ds(r0, OROWS_LAST)],
                        outh.at[pl.ds(r0, OROWS_LAST), pl.ds(cbase, HALF)])


@jax.jit
def _sc_call(x0, x1, idxj, idxi):
    mesh = plsc.VectorSubcoreMesh(core_axis_name="c", subcore_axis_name="s")
    return pl.kernel(
        _sc_body,
        out_type=jax.ShapeDtypeStruct((N_NODES, D_FEAT), jnp.float32),
        mesh=mesh,
        compiler_params=pltpu.CompilerParams(use_tc_tiling_on_sc=False),
        scratch_types=[
            pltpu.VMEM((PEDGES,), jnp.int32),
            pltpu.VMEM((NG, CHUNK), jnp.int32),
            pltpu.VMEM((NBUF * CHUNK, HALF), jnp.float32),
            pltpu.VMEM((16, HALF), jnp.float32),
            pltpu.VMEM_SHARED((XS_ROWS, HALF), jnp.float32),
            pltpu.VMEM_SHARED((ACC_ROWS, HALF), jnp.float32),
        ] + [pltpu.SemaphoreType.DMA] * 10,
    )(x0, x1, idxj, idxi)


def kernel(x, a3_indices, e3):
    del e3
    idx_j = a3_indices[:, 1]
    idx_i = a3_indices[:, 2]
    pad = BP - N_EDGES
    idx_j = jnp.concatenate([idx_j, jnp.zeros((pad,), jnp.int32)])
    idx_i = jnp.concatenate(
        [idx_i, jnp.full((pad,), N_NODES, jnp.int32)]).reshape(NSCHUNKS, CHUNK)
    return _sc_call(x[:, :HALF], x[:, HALF:], idx_j, idx_i)
